# baseline (speedup 1.0000x reference)
import functools

import jax
import jax.numpy as jnp
from jax import lax
from jax.experimental import pallas as pl
from jax.experimental.pallas import tpu as pltpu
from jax.experimental.pallas import tpu_sc as plsc

N = 10000
D_FEAT = 128
HID = 64
HEADS = 8
OUT = 10
H16 = 16
NPAD = 10240
NC, NS = 2, 16
NW = NC * NS
ROWS_PT = NPAD // NS
K = 80
CH = 129
E_PAD = NW * CH * K
E_TOT = 320000 + N
G = 4

_f32 = jnp.float32
_mesh = plsc.VectorSubcoreMesh(core_axis_name="c", subcore_axis_name="s")
_sc_params = pltpu.CompilerParams(use_tc_tiling_on_sc=False)


def _alpha_tabs(h, bn, asv, adv, as_ref, ad_ref):
    hr = h.reshape(bn, HEADS, HID)
    a_s = jnp.sum(hr * asv, axis=2)
    a_d = jnp.sum(hr * adv, axis=2)
    z = jnp.zeros((bn, H16 - HEADS), _f32)
    as_ref[...] = jnp.concatenate([a_s, z], axis=1)
    ad_ref[...] = jnp.concatenate([a_d, z], axis=1)


def _mm1_body(x_ref, w_ref, asv_ref, adv_ref,
              h0_ref, h1_ref, h2_ref, h3_ref, as_ref, ad_ref):
    h = jnp.dot(x_ref[...], w_ref[...], preferred_element_type=_f32,
                precision=lax.Precision.HIGHEST)
    for g, ref in enumerate((h0_ref, h1_ref, h2_ref, h3_ref)):
        ref[...] = h[:, g * 128:(g + 1) * 128]
    _alpha_tabs(h, h.shape[0], asv_ref[...], adv_ref[...], as_ref, ad_ref)


def _norm_relu(aa_ref, ab_ref, da_ref, db_ref, b_ref):
    rd = 1.0 / (da_ref[...] + db_ref[...] + 1e-16)
    b = b_ref[...]
    bn = rd.shape[0]
    parts = []
    for g in range(G):
        acc = aa_ref[g] + ab_ref[g]
        f = jnp.concatenate(
            [jnp.broadcast_to(rd[:, 2 * g:2 * g + 1], (bn, HID)),
             jnp.broadcast_to(rd[:, 2 * g + 1:2 * g + 2], (bn, HID))], axis=1)
        parts.append(jax.nn.relu(acc * f + b[0, g * 128:(g + 1) * 128]))
    return jnp.concatenate(parts, axis=1)


def _mm23_body(aa_ref, ab_ref, da_ref, db_ref, b_ref, x_ref, wa_ref, wb_ref,
               asv_ref, adv_ref,
               h0_ref, h1_ref, h2_ref, h3_ref, as_ref, ad_ref):
    a = _norm_relu(aa_ref, ab_ref, da_ref, db_ref, b_ref)
    h = (jnp.dot(a, wa_ref[...], preferred_element_type=_f32,
                 precision=lax.Precision.HIGHEST)
         + jnp.dot(x_ref[...], wb_ref[...], preferred_element_type=_f32,
                   precision=lax.Precision.HIGHEST))
    for g, ref in enumerate((h0_ref, h1_ref, h2_ref, h3_ref)):
        ref[...] = h[:, g * 128:(g + 1) * 128]
    _alpha_tabs(h, h.shape[0], asv_ref[...], adv_ref[...], as_ref, ad_ref)


def _mm3_body(aa_ref, ab_ref, da_ref, db_ref, b_ref, x_ref, wa_ref, wb_ref,
              asv_ref, adv_ref, h_ref, as_ref, ad_ref):
    a = _norm_relu(aa_ref, ab_ref, da_ref, db_ref, b_ref)
    h = (jnp.dot(a, wa_ref[...], preferred_element_type=_f32,
                 precision=lax.Precision.HIGHEST)
         + jnp.dot(x_ref[...], wb_ref[...], preferred_element_type=_f32,
                   precision=lax.Precision.HIGHEST))
    bn = h.shape[0]
    h_ref[...] = h
    a_s = jnp.sum(h * asv_ref[...], axis=1, keepdims=True)
    a_d = jnp.sum(h * adv_ref[...], axis=1, keepdims=True)
    z = jnp.zeros((bn, H16 - 1), _f32)
    as_ref[...] = jnp.concatenate([a_s, z], axis=1)
    ad_ref[...] = jnp.concatenate([a_d, z], axis=1)


def _epi3_body(aa_ref, ab_ref, da_ref, db_ref, b_ref, o_ref):
    rd = 1.0 / (da_ref[...] + db_ref[...] + 1e-16)
    s = (aa_ref[...] + ab_ref[...]) * rd[:, 0:1] + b_ref[...]
    bn = s.shape[0]
    col = lax.broadcasted_iota(jnp.int32, (bn, H16), 1)
    valid = col < OUT
    s = jnp.where(valid, s, -1e30)
    m = jnp.max(s, axis=1, keepdims=True)
    e = jnp.where(valid, jnp.exp(s - m), 0.0)
    lse = jnp.log(jnp.sum(e, axis=1, keepdims=True))
    o_ref[...] = (s - m - lse)[:, :OUT]


_BN = 1000
_spec_n16 = pl.BlockSpec((_BN, H16), lambda i: (i, 0))
_spec_acc = pl.BlockSpec((G, _BN, 128), lambda i: (0, i, 0))


def _mm1(x, w, asv, adv):
    return pl.pallas_call(
        _mm1_body,
        grid=(N // _BN,),
        in_specs=[
            pl.BlockSpec((_BN, D_FEAT), lambda i: (i, 0)),
            pl.BlockSpec((D_FEAT, HEADS * HID), lambda i: (0, 0)),
            pl.BlockSpec((1, HEADS, HID), lambda i: (0, 0, 0)),
            pl.BlockSpec((1, HEADS, HID), lambda i: (0, 0, 0)),
        ],
        out_specs=[pl.BlockSpec((_BN, 128), lambda i: (i, 0))] * 4
        + [_spec_n16] * 2,
        out_shape=[jax.ShapeDtypeStruct((N, 128), _f32)] * 4
        + [jax.ShapeDtypeStruct((N, H16), _f32)] * 2,
    )(x, w, asv, adv)


def _mm23(aa, ab, da, db, b, x, wa, wb, asv, adv):
    return pl.pallas_call(
        _mm23_body,
        grid=(N // _BN,),
        in_specs=[
            _spec_acc, _spec_acc, _spec_n16, _spec_n16,
            pl.BlockSpec((1, HEADS * HID), lambda i: (0, 0)),
            pl.BlockSpec((_BN, D_FEAT), lambda i: (i, 0)),
            pl.BlockSpec((HEADS * HID, HEADS * HID), lambda i: (0, 0)),
            pl.BlockSpec((D_FEAT, HEADS * HID), lambda i: (0, 0)),
            pl.BlockSpec((1, HEADS, HID), lambda i: (0, 0, 0)),
            pl.BlockSpec((1, HEADS, HID), lambda i: (0, 0, 0)),
        ],
        out_specs=[pl.BlockSpec((_BN, 128), lambda i: (i, 0))] * 4
        + [_spec_n16] * 2,
        out_shape=[jax.ShapeDtypeStruct((N, 128), _f32)] * 4
        + [jax.ShapeDtypeStruct((N, H16), _f32)] * 2,
    )(aa, ab, da, db, b, x, wa, wb, asv, adv)


def _mm3(aa, ab, da, db, b, x, wa, wb, asv, adv):
    return pl.pallas_call(
        _mm3_body,
        grid=(N // _BN,),
        in_specs=[
            _spec_acc, _spec_acc, _spec_n16, _spec_n16,
            pl.BlockSpec((1, HEADS * HID), lambda i: (0, 0)),
            pl.BlockSpec((_BN, D_FEAT), lambda i: (i, 0)),
            pl.BlockSpec((HEADS * HID, H16), lambda i: (0, 0)),
            pl.BlockSpec((D_FEAT, H16), lambda i: (0, 0)),
            pl.BlockSpec((1, H16), lambda i: (0, 0)),
            pl.BlockSpec((1, H16), lambda i: (0, 0)),
        ],
        out_specs=[_spec_n16] * 3,
        out_shape=[jax.ShapeDtypeStruct((N, H16), _f32)] * 3,
    )(aa, ab, da, db, b, x, wa, wb, asv, adv)


def _epi3(aa, ab, da, db, b):
    return pl.pallas_call(
        _epi3_body,
        grid=(N // _BN,),
        in_specs=[_spec_n16, _spec_n16, _spec_n16, _spec_n16,
                  pl.BlockSpec((1, H16), lambda i: (0, 0))],
        out_specs=pl.BlockSpec((_BN, OUT), lambda i: (i, 0)),
        out_shape=jax.ShapeDtypeStruct((N, OUT), _f32),
    )(aa, ab, da, db, b)


def _zero_rows(buf, width_vregs):
    z = jnp.zeros((16,), _f32)

    def row(e, _):
        for j in range(width_vregs):
            buf[e, pl.ds(j * 16, 16)] = z
        return 0

    lax.fori_loop(0, K, row, 0)


@functools.partial(
    pl.kernel,
    out_type=(
        jax.ShapeDtypeStruct((E_PAD, H16), _f32),
        jax.ShapeDtypeStruct((NPAD, H16), _f32),
        jax.ShapeDtypeStruct((NPAD, H16), _f32),
    ),
    mesh=_mesh,
    compiler_params=_sc_params,
    scratch_types=[
        pltpu.VMEM((CH, K), jnp.int32),
        pltpu.VMEM((CH, K), jnp.int32),
        pltpu.VMEM((2, K, H16), _f32),
        pltpu.VMEM((2, K, H16), _f32),
        pltpu.VMEM((2, K, H16), _f32),
        pltpu.VMEM((K, H16), _f32),
        pltpu.VMEM_SHARED((NPAD, H16), _f32),
        pltpu.SemaphoreType.DMA,
        pltpu.SemaphoreType.DMA,
        pltpu.SemaphoreType.DMA,
        pltpu.SemaphoreType.DMA,
    ],
)
def _phase1(src3_h, dst3_h, as_h, ad_h, ex_h, da_h, db_h,
            src_a, dst_a, as_v, ad_v, exo_v, z16, den_sh,
            gsem0, gsem1, wsem0, wsem1):
    cid = lax.axis_index("c")
    sid = lax.axis_index("s")
    wid = sid * NC + cid
    row0 = sid * ROWS_PT

    pltpu.sync_copy(src3_h.at[wid], src_a)
    pltpu.sync_copy(dst3_h.at[wid], dst_a)
    _zero_rows(z16, 1)
    for t in range(ROWS_PT // K):
        pltpu.sync_copy(z16, den_sh.at[pl.ds(row0 + t * K, K), :])
    plsc.subcore_barrier()

    def issue(j, slot, gsem):
        pltpu.async_copy(as_h.at[src_a.at[j]], as_v.at[slot], gsem)
        pltpu.async_copy(ad_h.at[dst_a.at[j]], ad_v.at[slot], gsem)

    def drain_g(slot, gsem):
        pltpu.make_async_copy(as_h.at[pl.ds(0, K)], as_v.at[slot], gsem).wait()
        pltpu.make_async_copy(ad_h.at[pl.ds(0, K)], ad_v.at[slot], gsem).wait()

    def drain_w(slot, wsem):
        pltpu.make_async_copy(exo_v.at[slot], ex_h.at[pl.ds(0, K), :], wsem).wait()

    issue(0, 0, gsem0)

    def step(i, _):
        p = i % 2
        nxt = i + 1

        @pl.when(jnp.logical_and(nxt < CH, nxt % 2 == 0))
        def _():
            drain_w(0, wsem0)
            issue(nxt, 0, gsem0)

        @pl.when(jnp.logical_and(nxt < CH, nxt % 2 == 1))
        def _():
            @pl.when(nxt >= 3)
            def _():
                drain_w(1, wsem1)

            issue(nxt, 1, gsem1)

        @pl.when(p == 0)
        def _():
            drain_g(0, gsem0)

        @pl.when(p == 1)
        def _():
            drain_g(1, gsem1)

        def edge(e, _):
            v = as_v[p, e, :] + ad_v[p, e, :]
            v = jnp.where(v > 0.0, v, 0.2 * v)
            exo_v[p, e, :] = jnp.exp(v)
            return 0

        lax.fori_loop(0, K, edge, 0, unroll=2)

        @pl.when(p == 0)
        def _():
            pltpu.async_copy(exo_v.at[0], ex_h.at[pl.ds((wid * CH + i) * K, K), :], wsem0)

        @pl.when(p == 1)
        def _():
            pltpu.async_copy(exo_v.at[1], ex_h.at[pl.ds((wid * CH + i) * K, K), :], wsem1)

        pltpu.sync_copy(exo_v.at[p], den_sh.at[dst_a.at[i]], add=True)
        return 0

    lax.fori_loop(0, CH, step, 0)
    drain_w(0, wsem0)
    drain_w(1, wsem1)
    plsc.subcore_barrier()

    @pl.when(cid == 0)
    def _():
        pltpu.sync_copy(den_sh.at[pl.ds(row0, ROWS_PT), :],
                        da_h.at[pl.ds(row0, ROWS_PT), :])

    @pl.when(cid == 1)
    def _():
        pltpu.sync_copy(den_sh.at[pl.ds(row0, ROWS_PT), :],
                        db_h.at[pl.ds(row0, ROWS_PT), :])


@functools.partial(
    pl.kernel,
    out_type=(
        jax.ShapeDtypeStruct((G, NPAD, 128), _f32),
        jax.ShapeDtypeStruct((G, NPAD, 128), _f32),
    ),
    mesh=_mesh,
    compiler_params=_sc_params,
    scratch_types=[
        pltpu.VMEM((2, K), jnp.int32),
        pltpu.VMEM((CH, K), jnp.int32),
        pltpu.VMEM((3, K, H16), _f32),
        pltpu.VMEM((3, K, 128), _f32),
        pltpu.VMEM_SHARED((NPAD, 128), _f32),
        pltpu.SemaphoreType.DMA,
        pltpu.SemaphoreType.DMA,
        pltpu.SemaphoreType.DMA,
        pltpu.SemaphoreType.DMA,
        pltpu.SemaphoreType.DMA,
        pltpu.SemaphoreType.DMA,
        pltpu.SemaphoreType.DMA,
        pltpu.SemaphoreType.DMA,
    ],
)
def _phase2(src3_h, dst3_h, ex_h, zr_h, h0_h, h1_h, h2_h, h3_h, pa_h, pb_h,
            src_s, dst_a, ex_v, h_v, acc_sh,
            isem0, isem1, gsem0, gsem1, gsem2, ssem0, ssem1, ssem2):
    cid = lax.axis_index("c")
    sid = lax.axis_index("s")
    wid = sid * NC + cid
    row0 = sid * ROWS_PT
    GS = (gsem0, gsem1, gsem2)
    SS = (ssem0, ssem1, ssem2)

    pltpu.sync_copy(dst3_h.at[wid], dst_a)

    def idx_issue(j, slot, isem):
        pltpu.async_copy(src3_h.at[wid, j], src_s.at[slot], isem)

    def idx_drain(slot, isem):
        pltpu.make_async_copy(src3_h.at[0, 0], src_s.at[slot], isem).wait()

    for g, h_tab in enumerate((h0_h, h1_h, h2_h, h3_h)):
        for t in range(ROWS_PT // K):
            pltpu.sync_copy(zr_h, acc_sh.at[pl.ds(row0 + t * K, K), :])
        plsc.subcore_barrier()

        def issue(j, slot, gsem):
            pltpu.async_copy(
                ex_h.at[pl.ds((wid * CH + j) * K, K), :], ex_v.at[slot], gsem)
            pltpu.async_copy(h_tab.at[src_s.at[j % 2]], h_v.at[slot], gsem)

        def drain_g(slot, gsem):
            pltpu.make_async_copy(ex_h.at[pl.ds(0, K), :], ex_v.at[slot],
                                  gsem).wait()
            pltpu.make_async_copy(h_tab.at[pl.ds(0, K)], h_v.at[slot],
                                  gsem).wait()

        def drain_s(slot, ssem):
            pltpu.make_async_copy(h_v.at[slot], acc_sh.at[pl.ds(0, K), :],
                                  ssem).wait()

        idx_issue(0, 0, isem0)
        idx_issue(1, 1, isem1)
        idx_drain(0, isem0)
        issue(0, 0, gsem0)

        def step(i, _):
            s = i % 3
            sn = (i + 1) % 3
            nxt = i + 1

            for c in range(3):
                pl.when(s == c)(lambda c=c: drain_g(c, GS[c]))

            def edge(e, _):
                exr = ex_v[s, e, :]
                c0 = exr[2 * g]
                c1 = exr[2 * g + 1]
                for j in range(4):
                    h_v[s, e, pl.ds(j * 16, 16)] = \
                        h_v[s, e, pl.ds(j * 16, 16)] * c0
                for j in range(4, 8):
                    h_v[s, e, pl.ds(j * 16, 16)] = \
                        h_v[s, e, pl.ds(j * 16, 16)] * c1
                return 0

            lax.fori_loop(0, K, edge, 0, unroll=2)

            for c in range(3):
                pl.when(jnp.logical_and(sn == c, i >= 2))(
                    lambda c=c: drain_s(c, SS[c]))

            @pl.when(jnp.logical_and(nxt < CH, nxt % 2 == 0))
            def _():
                idx_drain(0, isem0)

            @pl.when(jnp.logical_and(nxt < CH, nxt % 2 == 1))
            def _():
                idx_drain(1, isem1)

            for c in range(3):
                pl.when(jnp.logical_and(nxt < CH, sn == c))(
                    lambda c=c: issue(nxt, c, GS[c]))

            for c in range(3):
                def _scat(c=c):
                    pltpu.async_copy(h_v.at[c], acc_sh.at[dst_a.at[i]],
                                     SS[c], add=True)

                pl.when(s == c)(_scat)

            @pl.when(jnp.logical_and(i + 2 < CH, i % 2 == 0))
            def _():
                idx_issue(i + 2, 0, isem0)

            @pl.when(jnp.logical_and(i + 2 < CH, i % 2 == 1))
            def _():
                idx_issue(i + 2, 1, isem1)

            return 0

        lax.fori_loop(0, CH, step, 0)
        drain_s((CH - 2) % 3, SS[(CH - 2) % 3])
        drain_s((CH - 1) % 3, SS[(CH - 1) % 3])
        plsc.subcore_barrier()

        @pl.when(cid == 0)
        def _():
            pltpu.sync_copy(acc_sh.at[pl.ds(row0, ROWS_PT), :],
                            pa_h.at[g, pl.ds(row0, ROWS_PT), :])

        @pl.when(cid == 1)
        def _():
            pltpu.sync_copy(acc_sh.at[pl.ds(row0, ROWS_PT), :],
                            pb_h.at[g, pl.ds(row0, ROWS_PT), :])

        plsc.subcore_barrier()


@functools.partial(
    pl.kernel,
    out_type=(
        jax.ShapeDtypeStruct((NPAD, H16), _f32),
        jax.ShapeDtypeStruct((NPAD, H16), _f32),
    ),
    mesh=_mesh,
    compiler_params=_sc_params,
    scratch_types=[
        pltpu.VMEM((CH, K), jnp.int32),
        pltpu.VMEM((CH, K), jnp.int32),
        pltpu.VMEM((2, K, H16), _f32),
        pltpu.VMEM((2, K, H16), _f32),
        pltpu.VMEM((K, H16), _f32),
        pltpu.VMEM_SHARED((NPAD, H16), _f32),
        pltpu.SemaphoreType.DMA,
        pltpu.SemaphoreType.DMA,
    ],
)
def _phase2s(src3_h, dst3_h, ex_h, ht_h, pa_h, pb_h,
             src_a, dst_a, ex_v, h_v, z16, acc_sh, gsem0, gsem1):
    cid = lax.axis_index("c")
    sid = lax.axis_index("s")
    wid = sid * NC + cid
    row0 = sid * ROWS_PT

    pltpu.sync_copy(src3_h.at[wid], src_a)
    pltpu.sync_copy(dst3_h.at[wid], dst_a)
    _zero_rows(z16, 1)
    for t in range(ROWS_PT // K):
        pltpu.sync_copy(z16, acc_sh.at[pl.ds(row0 + t * K, K), :])
    plsc.subcore_barrier()

    def issue(j, slot, gsem):
        pltpu.async_copy(ex_h.at[pl.ds((wid * CH + j) * K, K), :], ex_v.at[slot], gsem)
        pltpu.async_copy(ht_h.at[src_a.at[j]], h_v.at[slot], gsem)

    def drain_g(slot, gsem):
        pltpu.make_async_copy(ex_h.at[pl.ds(0, K), :], ex_v.at[slot], gsem).wait()
        pltpu.make_async_copy(ht_h.at[pl.ds(0, K)], h_v.at[slot], gsem).wait()

    issue(0, 0, gsem0)

    def step(i, _):
        p = i % 2
        nxt = i + 1

        @pl.when(jnp.logical_and(nxt < CH, nxt % 2 == 0))
        def _():
            issue(nxt, 0, gsem0)

        @pl.when(jnp.logical_and(nxt < CH, nxt % 2 == 1))
        def _():
            issue(nxt, 1, gsem1)

        @pl.when(p == 0)
        def _():
            drain_g(0, gsem0)

        @pl.when(p == 1)
        def _():
            drain_g(1, gsem1)

        def edge(e, _):
            exr = ex_v[p, e, :]
            h_v[p, e, :] = h_v[p, e, :] * exr[0]
            return 0

        lax.fori_loop(0, K, edge, 0, unroll=2)
        pltpu.sync_copy(h_v.at[p], acc_sh.at[dst_a.at[i]], add=True)
        return 0

    lax.fori_loop(0, CH, step, 0)
    plsc.subcore_barrier()

    @pl.when(cid == 0)
    def _():
        pltpu.sync_copy(acc_sh.at[pl.ds(row0, ROWS_PT), :],
                        pa_h.at[pl.ds(row0, ROWS_PT), :])

    @pl.when(cid == 1)
    def _():
        pltpu.sync_copy(acc_sh.at[pl.ds(row0, ROWS_PT), :],
                        pb_h.at[pl.ds(row0, ROWS_PT), :])


def _pad_tab(t):
    return jnp.concatenate([t, jnp.zeros((NPAD - N, H16), _f32)], axis=0)


def kernel(x, edge_index, W1, att_src1, att_dst1, b1,
           W2, att_src2, att_dst2, b2, W3, att_src3, att_dst3, b3):
    loop = jnp.arange(N, dtype=edge_index.dtype)
    npad_e = E_PAD - E_TOT
    src = jnp.concatenate(
        [edge_index[0], loop,
         jnp.zeros((npad_e,), edge_index.dtype)]).reshape(NW, CH, K)
    dst = jnp.concatenate(
        [edge_index[1], loop,
         jnp.full((npad_e,), N, edge_index.dtype)]).reshape(NW, CH, K)

    zr = jnp.zeros((K, 128), _f32)

    h0, h1, h2, h3, a_s, a_d = _mm1(x, W1, att_src1, att_dst1)
    ex, da, db = _phase1(src, dst, _pad_tab(a_s), _pad_tab(a_d))
    pa, pb = _phase2(src, dst, ex, zr, h0, h1, h2, h3)

    h0, h1, h2, h3, a_s, a_d = _mm23(
        pa, pb, da, db, b1.reshape(1, -1), x, W2[:HEADS * HID],
        W2[HEADS * HID:], att_src2, att_dst2)
    ex, da, db = _phase1(src, dst, _pad_tab(a_s), _pad_tab(a_d))
    pa, pb = _phase2(src, dst, ex, zr, h0, h1, h2, h3)

    w3p = jnp.concatenate([W3, jnp.zeros((W3.shape[0], H16 - OUT), _f32)],
                          axis=1)
    a3s = jnp.concatenate(
        [att_src3.reshape(1, OUT), jnp.zeros((1, H16 - OUT), _f32)], axis=1)
    a3d = jnp.concatenate(
        [att_dst3.reshape(1, OUT), jnp.zeros((1, H16 - OUT), _f32)], axis=1)
    ht, a_s, a_d = _mm3(
        pa, pb, da, db, b2.reshape(1, -1), x, w3p[:HEADS * HID],
        w3p[HEADS * HID:], a3s, a3d)
    ex, da, db = _phase1(src, dst, _pad_tab(a_s), _pad_tab(a_d))
    pa3, pb3 = _phase2s(src, dst, ex, ht)

    b3p = jnp.concatenate(
        [b3.reshape(1, OUT), jnp.zeros((1, H16 - OUT), _f32)], axis=1)
    return _epi3(pa3, pb3, da, db, b3p)


# speedup vs baseline: 1.8632x; 1.8632x over previous
import functools

import jax
import jax.numpy as jnp
from jax import lax
from jax.experimental import pallas as pl
from jax.experimental.pallas import tpu as pltpu
from jax.experimental.pallas import tpu_sc as plsc

N = 10000
D_FEAT = 128
HID = 64
HEADS = 8
OUT = 10
H16 = 16
NPAD = 10240
NC, NS = 2, 16
NW = NC * NS
ROWS_PT = NPAD // NS
K = 128
CH = 81
E_PAD = NW * CH * K
E_TOT = 320000 + N
G = 4

_f32 = jnp.float32
_mesh = plsc.VectorSubcoreMesh(core_axis_name="c", subcore_axis_name="s")
_sc_params = pltpu.CompilerParams(use_tc_tiling_on_sc=False)


def _alpha_tabs(h, bn, asv, adv, as_ref, ad_ref):
    hr = h.reshape(bn, HEADS, HID)
    a_s = jnp.sum(hr * asv, axis=2)
    a_d = jnp.sum(hr * adv, axis=2)
    z = jnp.zeros((bn, H16 - HEADS), _f32)
    as_ref[...] = jnp.concatenate([a_s, z], axis=1)
    ad_ref[...] = jnp.concatenate([a_d, z], axis=1)


def _mm1_body(x_ref, w_ref, asv_ref, adv_ref,
              h0_ref, h1_ref, h2_ref, h3_ref, as_ref, ad_ref):
    h = jnp.dot(x_ref[...], w_ref[...], preferred_element_type=_f32,
                precision=lax.Precision.HIGHEST)
    for g, ref in enumerate((h0_ref, h1_ref, h2_ref, h3_ref)):
        ref[...] = h[:, g * 128:(g + 1) * 128]
    _alpha_tabs(h, h.shape[0], asv_ref[...], adv_ref[...], as_ref, ad_ref)


def _norm_relu(aa_ref, ab_ref, da_ref, db_ref, b_ref):
    rd = 1.0 / (da_ref[...] + db_ref[...] + 1e-16)
    b = b_ref[...]
    bn = rd.shape[0]
    parts = []
    for g in range(G):
        acc = aa_ref[g] + ab_ref[g]
        f = jnp.concatenate(
            [jnp.broadcast_to(rd[:, 2 * g:2 * g + 1], (bn, HID)),
             jnp.broadcast_to(rd[:, 2 * g + 1:2 * g + 2], (bn, HID))], axis=1)
        parts.append(jax.nn.relu(acc * f + b[0, g * 128:(g + 1) * 128]))
    return jnp.concatenate(parts, axis=1)


def _mm23_body(aa_ref, ab_ref, da_ref, db_ref, b_ref, x_ref, wa_ref, wb_ref,
               asv_ref, adv_ref,
               h0_ref, h1_ref, h2_ref, h3_ref, as_ref, ad_ref):
    a = _norm_relu(aa_ref, ab_ref, da_ref, db_ref, b_ref)
    h = (jnp.dot(a, wa_ref[...], preferred_element_type=_f32,
                 precision=lax.Precision.HIGHEST)
         + jnp.dot(x_ref[...], wb_ref[...], preferred_element_type=_f32,
                   precision=lax.Precision.HIGHEST))
    for g, ref in enumerate((h0_ref, h1_ref, h2_ref, h3_ref)):
        ref[...] = h[:, g * 128:(g + 1) * 128]
    _alpha_tabs(h, h.shape[0], asv_ref[...], adv_ref[...], as_ref, ad_ref)


def _mm3_body(aa_ref, ab_ref, da_ref, db_ref, b_ref, x_ref, wa_ref, wb_ref,
              asv_ref, adv_ref, h_ref, as_ref, ad_ref):
    a = _norm_relu(aa_ref, ab_ref, da_ref, db_ref, b_ref)
    h = (jnp.dot(a, wa_ref[...], preferred_element_type=_f32,
                 precision=lax.Precision.HIGHEST)
         + jnp.dot(x_ref[...], wb_ref[...], preferred_element_type=_f32,
                   precision=lax.Precision.HIGHEST))
    bn = h.shape[0]
    h_ref[...] = h
    a_s = jnp.sum(h * asv_ref[...], axis=1, keepdims=True)
    a_d = jnp.sum(h * adv_ref[...], axis=1, keepdims=True)
    z = jnp.zeros((bn, H16 - 1), _f32)
    as_ref[...] = jnp.concatenate([a_s, z], axis=1)
    ad_ref[...] = jnp.concatenate([a_d, z], axis=1)


def _epi3_body(aa_ref, ab_ref, da_ref, db_ref, b_ref, o_ref):
    rd = 1.0 / (da_ref[...] + db_ref[...] + 1e-16)
    s = (aa_ref[...] + ab_ref[...]) * rd[:, 0:1] + b_ref[...]
    bn = s.shape[0]
    col = lax.broadcasted_iota(jnp.int32, (bn, H16), 1)
    valid = col < OUT
    s = jnp.where(valid, s, -1e30)
    m = jnp.max(s, axis=1, keepdims=True)
    e = jnp.where(valid, jnp.exp(s - m), 0.0)
    lse = jnp.log(jnp.sum(e, axis=1, keepdims=True))
    o_ref[...] = (s - m - lse)[:, :OUT]


_BN = 1000
_spec_n16 = pl.BlockSpec((_BN, H16), lambda i: (i, 0))
_spec_acc = pl.BlockSpec((G, _BN, 128), lambda i: (0, i, 0))


def _mm1(x, w, asv, adv):
    return pl.pallas_call(
        _mm1_body,
        grid=(N // _BN,),
        in_specs=[
            pl.BlockSpec((_BN, D_FEAT), lambda i: (i, 0)),
            pl.BlockSpec((D_FEAT, HEADS * HID), lambda i: (0, 0)),
            pl.BlockSpec((1, HEADS, HID), lambda i: (0, 0, 0)),
            pl.BlockSpec((1, HEADS, HID), lambda i: (0, 0, 0)),
        ],
        out_specs=[pl.BlockSpec((_BN, 128), lambda i: (i, 0))] * 4
        + [_spec_n16] * 2,
        out_shape=[jax.ShapeDtypeStruct((N, 128), _f32)] * 4
        + [jax.ShapeDtypeStruct((N, H16), _f32)] * 2,
    )(x, w, asv, adv)


def _mm23(aa, ab, da, db, b, x, wa, wb, asv, adv):
    return pl.pallas_call(
        _mm23_body,
        grid=(N // _BN,),
        in_specs=[
            _spec_acc, _spec_acc, _spec_n16, _spec_n16,
            pl.BlockSpec((1, HEADS * HID), lambda i: (0, 0)),
            pl.BlockSpec((_BN, D_FEAT), lambda i: (i, 0)),
            pl.BlockSpec((HEADS * HID, HEADS * HID), lambda i: (0, 0)),
            pl.BlockSpec((D_FEAT, HEADS * HID), lambda i: (0, 0)),
            pl.BlockSpec((1, HEADS, HID), lambda i: (0, 0, 0)),
            pl.BlockSpec((1, HEADS, HID), lambda i: (0, 0, 0)),
        ],
        out_specs=[pl.BlockSpec((_BN, 128), lambda i: (i, 0))] * 4
        + [_spec_n16] * 2,
        out_shape=[jax.ShapeDtypeStruct((N, 128), _f32)] * 4
        + [jax.ShapeDtypeStruct((N, H16), _f32)] * 2,
    )(aa, ab, da, db, b, x, wa, wb, asv, adv)


def _mm3(aa, ab, da, db, b, x, wa, wb, asv, adv):
    return pl.pallas_call(
        _mm3_body,
        grid=(N // _BN,),
        in_specs=[
            _spec_acc, _spec_acc, _spec_n16, _spec_n16,
            pl.BlockSpec((1, HEADS * HID), lambda i: (0, 0)),
            pl.BlockSpec((_BN, D_FEAT), lambda i: (i, 0)),
            pl.BlockSpec((HEADS * HID, H16), lambda i: (0, 0)),
            pl.BlockSpec((D_FEAT, H16), lambda i: (0, 0)),
            pl.BlockSpec((1, H16), lambda i: (0, 0)),
            pl.BlockSpec((1, H16), lambda i: (0, 0)),
        ],
        out_specs=[_spec_n16] * 3,
        out_shape=[jax.ShapeDtypeStruct((N, H16), _f32)] * 3,
    )(aa, ab, da, db, b, x, wa, wb, asv, adv)


def _epi3(aa, ab, da, db, b):
    return pl.pallas_call(
        _epi3_body,
        grid=(N // _BN,),
        in_specs=[_spec_n16, _spec_n16, _spec_n16, _spec_n16,
                  pl.BlockSpec((1, H16), lambda i: (0, 0))],
        out_specs=pl.BlockSpec((_BN, OUT), lambda i: (i, 0)),
        out_shape=jax.ShapeDtypeStruct((N, OUT), _f32),
    )(aa, ab, da, db, b)


def _zero_rows(buf, width_vregs):
    z = jnp.zeros((16,), _f32)

    def row(e, _):
        for j in range(width_vregs):
            buf[e, pl.ds(j * 16, 16)] = z
        return 0

    lax.fori_loop(0, K, row, 0)


@functools.partial(
    pl.kernel,
    out_type=(
        jax.ShapeDtypeStruct((E_PAD, H16), _f32),
        jax.ShapeDtypeStruct((NPAD, H16), _f32),
        jax.ShapeDtypeStruct((NPAD, H16), _f32),
    ),
    mesh=_mesh,
    compiler_params=_sc_params,
    scratch_types=[
        pltpu.VMEM((CH, K), jnp.int32),
        pltpu.VMEM((CH, K), jnp.int32),
        pltpu.VMEM((2, K, H16), _f32),
        pltpu.VMEM((2, K, H16), _f32),
        pltpu.VMEM((2, K, H16), _f32),
        pltpu.VMEM((K, H16), _f32),
        pltpu.VMEM_SHARED((NPAD, H16), _f32),
        pltpu.SemaphoreType.DMA,
        pltpu.SemaphoreType.DMA,
        pltpu.SemaphoreType.DMA,
        pltpu.SemaphoreType.DMA,
    ],
)
def _phase1(src3_h, dst3_h, as_h, ad_h, ex_h, da_h, db_h,
            src_a, dst_a, as_v, ad_v, exo_v, z16, den_sh,
            gsem0, gsem1, wsem0, wsem1):
    cid = lax.axis_index("c")
    sid = lax.axis_index("s")
    wid = sid * NC + cid
    row0 = sid * ROWS_PT

    pltpu.sync_copy(src3_h.at[wid], src_a)
    pltpu.sync_copy(dst3_h.at[wid], dst_a)
    _zero_rows(z16, 1)
    for t in range(ROWS_PT // K):
        pltpu.sync_copy(z16, den_sh.at[pl.ds(row0 + t * K, K), :])
    plsc.subcore_barrier()

    def issue(j, slot, gsem):
        pltpu.async_copy(as_h.at[src_a.at[j]], as_v.at[slot], gsem)
        pltpu.async_copy(ad_h.at[dst_a.at[j]], ad_v.at[slot], gsem)

    def drain_g(slot, gsem):
        pltpu.make_async_copy(as_h.at[pl.ds(0, K)], as_v.at[slot], gsem).wait()
        pltpu.make_async_copy(ad_h.at[pl.ds(0, K)], ad_v.at[slot], gsem).wait()

    def drain_w(slot, wsem):
        pltpu.make_async_copy(exo_v.at[slot], ex_h.at[pl.ds(0, K), :], wsem).wait()

    issue(0, 0, gsem0)

    def step(i, _):
        p = i % 2
        nxt = i + 1

        @pl.when(jnp.logical_and(nxt < CH, nxt % 2 == 0))
        def _():
            drain_w(0, wsem0)
            issue(nxt, 0, gsem0)

        @pl.when(jnp.logical_and(nxt < CH, nxt % 2 == 1))
        def _():
            @pl.when(nxt >= 3)
            def _():
                drain_w(1, wsem1)

            issue(nxt, 1, gsem1)

        @pl.when(p == 0)
        def _():
            drain_g(0, gsem0)

        @pl.when(p == 1)
        def _():
            drain_g(1, gsem1)

        def edge(e, _):
            v = as_v[p, e, :] + ad_v[p, e, :]
            v = jnp.where(v > 0.0, v, 0.2 * v)
            exo_v[p, e, :] = jnp.exp(v)
            return 0

        lax.fori_loop(0, K, edge, 0, unroll=2)

        @pl.when(p == 0)
        def _():
            pltpu.async_copy(exo_v.at[0], ex_h.at[pl.ds((wid * CH + i) * K, K), :], wsem0)

        @pl.when(p == 1)
        def _():
            pltpu.async_copy(exo_v.at[1], ex_h.at[pl.ds((wid * CH + i) * K, K), :], wsem1)

        pltpu.sync_copy(exo_v.at[p], den_sh.at[dst_a.at[i]], add=True)
        return 0

    lax.fori_loop(0, CH, step, 0)
    drain_w(0, wsem0)
    drain_w(1, wsem1)
    plsc.subcore_barrier()

    @pl.when(cid == 0)
    def _():
        pltpu.sync_copy(den_sh.at[pl.ds(row0, ROWS_PT), :],
                        da_h.at[pl.ds(row0, ROWS_PT), :])

    @pl.when(cid == 1)
    def _():
        pltpu.sync_copy(den_sh.at[pl.ds(row0, ROWS_PT), :],
                        db_h.at[pl.ds(row0, ROWS_PT), :])


@functools.partial(
    pl.kernel,
    out_type=(
        jax.ShapeDtypeStruct((G, NPAD, 128), _f32),
        jax.ShapeDtypeStruct((G, NPAD, 128), _f32),
    ),
    mesh=_mesh,
    compiler_params=_sc_params,
    scratch_types=[
        pltpu.VMEM((2, K), jnp.int32),
        pltpu.VMEM((2, K), jnp.int32),
        pltpu.VMEM((2, K, H16), _f32),
        pltpu.VMEM((2, K, 128), _f32),
        pltpu.VMEM_SHARED((NPAD, 128), _f32),
        pltpu.SemaphoreType.DMA,
        pltpu.SemaphoreType.DMA,
        pltpu.SemaphoreType.DMA,
        pltpu.SemaphoreType.DMA,
    ],
)
def _phase2(src3_h, dst3_h, ex_h, zr_h, h0_h, h1_h, h2_h, h3_h, pa_h, pb_h,
            src_s, dst_s, ex_v, h_v, acc_sh, isem0, isem1, gsem0, gsem1):
    cid = lax.axis_index("c")
    sid = lax.axis_index("s")
    wid = sid * NC + cid
    row0 = sid * ROWS_PT

    def idx_issue(j, slot, isem):
        pltpu.async_copy(src3_h.at[wid, j], src_s.at[slot], isem)
        pltpu.async_copy(dst3_h.at[wid, j], dst_s.at[slot], isem)

    def idx_drain(slot, isem):
        pltpu.make_async_copy(src3_h.at[0, 0], src_s.at[slot], isem).wait()
        pltpu.make_async_copy(dst3_h.at[0, 0], dst_s.at[slot], isem).wait()

    for g, h_tab in enumerate((h0_h, h1_h, h2_h, h3_h)):
        for t in range(ROWS_PT // K):
            pltpu.sync_copy(zr_h, acc_sh.at[pl.ds(row0 + t * K, K), :])
        plsc.subcore_barrier()

        def issue(j, slot, gsem):
            pltpu.async_copy(
                ex_h.at[pl.ds((wid * CH + j) * K, K), :], ex_v.at[slot], gsem)
            pltpu.async_copy(h_tab.at[src_s.at[slot]], h_v.at[slot], gsem)

        def drain_g(slot, gsem):
            pltpu.make_async_copy(ex_h.at[pl.ds(0, K), :], ex_v.at[slot],
                                  gsem).wait()
            pltpu.make_async_copy(h_tab.at[pl.ds(0, K)], h_v.at[slot],
                                  gsem).wait()

        idx_issue(0, 0, isem0)
        idx_issue(1, 1, isem1)
        idx_drain(0, isem0)
        issue(0, 0, gsem0)

        def step(i, _):
            p = i % 2
            nxt = i + 1

            @pl.when(jnp.logical_and(nxt < CH, nxt % 2 == 0))
            def _():
                idx_drain(0, isem0)
                issue(nxt, 0, gsem0)

            @pl.when(jnp.logical_and(nxt < CH, nxt % 2 == 1))
            def _():
                idx_drain(1, isem1)
                issue(nxt, 1, gsem1)

            @pl.when(p == 0)
            def _():
                drain_g(0, gsem0)

            @pl.when(p == 1)
            def _():
                drain_g(1, gsem1)

            def edge(e, _):
                exr = ex_v[p, e, :]
                c0 = exr[2 * g]
                c1 = exr[2 * g + 1]
                for j in range(4):
                    h_v[p, e, pl.ds(j * 16, 16)] = \
                        h_v[p, e, pl.ds(j * 16, 16)] * c0
                for j in range(4, 8):
                    h_v[p, e, pl.ds(j * 16, 16)] = \
                        h_v[p, e, pl.ds(j * 16, 16)] * c1
                return 0

            lax.fori_loop(0, K, edge, 0, unroll=2)
            pltpu.sync_copy(h_v.at[p], acc_sh.at[dst_s.at[p]], add=True)

            @pl.when(jnp.logical_and(i + 2 < CH, p == 0))
            def _():
                idx_issue(i + 2, 0, isem0)

            @pl.when(jnp.logical_and(i + 2 < CH, p == 1))
            def _():
                idx_issue(i + 2, 1, isem1)

            return 0

        lax.fori_loop(0, CH, step, 0)
        plsc.subcore_barrier()

        @pl.when(cid == 0)
        def _():
            pltpu.sync_copy(acc_sh.at[pl.ds(row0, ROWS_PT), :],
                            pa_h.at[g, pl.ds(row0, ROWS_PT), :])

        @pl.when(cid == 1)
        def _():
            pltpu.sync_copy(acc_sh.at[pl.ds(row0, ROWS_PT), :],
                            pb_h.at[g, pl.ds(row0, ROWS_PT), :])

        plsc.subcore_barrier()


@functools.partial(
    pl.kernel,
    out_type=(
        jax.ShapeDtypeStruct((NPAD, H16), _f32),
        jax.ShapeDtypeStruct((NPAD, H16), _f32),
    ),
    mesh=_mesh,
    compiler_params=_sc_params,
    scratch_types=[
        pltpu.VMEM((CH, K), jnp.int32),
        pltpu.VMEM((CH, K), jnp.int32),
        pltpu.VMEM((2, K, H16), _f32),
        pltpu.VMEM((2, K, H16), _f32),
        pltpu.VMEM((K, H16), _f32),
        pltpu.VMEM_SHARED((NPAD, H16), _f32),
        pltpu.SemaphoreType.DMA,
        pltpu.SemaphoreType.DMA,
    ],
)
def _phase2s(src3_h, dst3_h, ex_h, ht_h, pa_h, pb_h,
             src_a, dst_a, ex_v, h_v, z16, acc_sh, gsem0, gsem1):
    cid = lax.axis_index("c")
    sid = lax.axis_index("s")
    wid = sid * NC + cid
    row0 = sid * ROWS_PT

    pltpu.sync_copy(src3_h.at[wid], src_a)
    pltpu.sync_copy(dst3_h.at[wid], dst_a)
    _zero_rows(z16, 1)
    for t in range(ROWS_PT // K):
        pltpu.sync_copy(z16, acc_sh.at[pl.ds(row0 + t * K, K), :])
    plsc.subcore_barrier()

    def issue(j, slot, gsem):
        pltpu.async_copy(ex_h.at[pl.ds((wid * CH + j) * K, K), :], ex_v.at[slot], gsem)
        pltpu.async_copy(ht_h.at[src_a.at[j]], h_v.at[slot], gsem)

    def drain_g(slot, gsem):
        pltpu.make_async_copy(ex_h.at[pl.ds(0, K), :], ex_v.at[slot], gsem).wait()
        pltpu.make_async_copy(ht_h.at[pl.ds(0, K)], h_v.at[slot], gsem).wait()

    issue(0, 0, gsem0)

    def step(i, _):
        p = i % 2
        nxt = i + 1

        @pl.when(jnp.logical_and(nxt < CH, nxt % 2 == 0))
        def _():
            issue(nxt, 0, gsem0)

        @pl.when(jnp.logical_and(nxt < CH, nxt % 2 == 1))
        def _():
            issue(nxt, 1, gsem1)

        @pl.when(p == 0)
        def _():
            drain_g(0, gsem0)

        @pl.when(p == 1)
        def _():
            drain_g(1, gsem1)

        def edge(e, _):
            exr = ex_v[p, e, :]
            h_v[p, e, :] = h_v[p, e, :] * exr[0]
            return 0

        lax.fori_loop(0, K, edge, 0, unroll=2)
        pltpu.sync_copy(h_v.at[p], acc_sh.at[dst_a.at[i]], add=True)
        return 0

    lax.fori_loop(0, CH, step, 0)
    plsc.subcore_barrier()

    @pl.when(cid == 0)
    def _():
        pltpu.sync_copy(acc_sh.at[pl.ds(row0, ROWS_PT), :],
                        pa_h.at[pl.ds(row0, ROWS_PT), :])

    @pl.when(cid == 1)
    def _():
        pltpu.sync_copy(acc_sh.at[pl.ds(row0, ROWS_PT), :],
                        pb_h.at[pl.ds(row0, ROWS_PT), :])


def _pad_tab(t):
    return jnp.concatenate([t, jnp.zeros((NPAD - N, H16), _f32)], axis=0)


def kernel(x, edge_index, W1, att_src1, att_dst1, b1,
           W2, att_src2, att_dst2, b2, W3, att_src3, att_dst3, b3):
    loop = jnp.arange(N, dtype=edge_index.dtype)
    npad_e = E_PAD - E_TOT
    src = jnp.concatenate(
        [edge_index[0], loop,
         jnp.zeros((npad_e,), edge_index.dtype)]).reshape(NW, CH, K)
    dst = jnp.concatenate(
        [edge_index[1], loop,
         jnp.full((npad_e,), N, edge_index.dtype)]).reshape(NW, CH, K)

    zr = jnp.zeros((K, 128), _f32)

    h0, h1, h2, h3, a_s, a_d = _mm1(x, W1, att_src1, att_dst1)
    ex, da, db = _phase1(src, dst, _pad_tab(a_s), _pad_tab(a_d))
    pa, pb = _phase2(src, dst, ex, zr, h0, h1, h2, h3)

    h0, h1, h2, h3, a_s, a_d = _mm23(
        pa, pb, da, db, b1.reshape(1, -1), x, W2[:HEADS * HID],
        W2[HEADS * HID:], att_src2, att_dst2)
    ex, da, db = _phase1(src, dst, _pad_tab(a_s), _pad_tab(a_d))
    pa, pb = _phase2(src, dst, ex, zr, h0, h1, h2, h3)

    w3p = jnp.concatenate([W3, jnp.zeros((W3.shape[0], H16 - OUT), _f32)],
                          axis=1)
    a3s = jnp.concatenate(
        [att_src3.reshape(1, OUT), jnp.zeros((1, H16 - OUT), _f32)], axis=1)
    a3d = jnp.concatenate(
        [att_dst3.reshape(1, OUT), jnp.zeros((1, H16 - OUT), _f32)], axis=1)
    ht, a_s, a_d = _mm3(
        pa, pb, da, db, b2.reshape(1, -1), x, w3p[:HEADS * HID],
        w3p[HEADS * HID:], a3s, a3d)
    ex, da, db = _phase1(src, dst, _pad_tab(a_s), _pad_tab(a_d))
    pa3, pb3 = _phase2s(src, dst, ex, ht)

    b3p = jnp.concatenate(
        [b3.reshape(1, OUT), jnp.zeros((1, H16 - OUT), _f32)], axis=1)
    return _epi3(pa3, pb3, da, db, b3p)


# speedup vs baseline: 1.8695x; 1.0034x over previous
import functools

import jax
import jax.numpy as jnp
from jax import lax
from jax.experimental import pallas as pl
from jax.experimental.pallas import tpu as pltpu
from jax.experimental.pallas import tpu_sc as plsc

N = 10000
D_FEAT = 128
HID = 64
HEADS = 8
OUT = 10
H16 = 16
NPAD = 10240
NC, NS = 2, 16
NW = NC * NS
ROWS_PT = NPAD // NS
K = 128
CH = 81
E_PAD = NW * CH * K
E_TOT = 320000 + N
G = 4

_f32 = jnp.float32
_mesh = plsc.VectorSubcoreMesh(core_axis_name="c", subcore_axis_name="s")
_sc_params = pltpu.CompilerParams(use_tc_tiling_on_sc=False)


def _alpha_tabs(h, bn, asv, adv, as_ref, ad_ref):
    hr = h.reshape(bn, HEADS, HID)
    a_s = jnp.sum(hr * asv, axis=2)
    a_d = jnp.sum(hr * adv, axis=2)
    z = jnp.zeros((bn, H16 - HEADS), _f32)
    as_ref[...] = jnp.concatenate([a_s, z], axis=1)
    ad_ref[...] = jnp.concatenate([a_d, z], axis=1)


def _mm1_body(x_ref, w_ref, asv_ref, adv_ref,
              h0_ref, h1_ref, h2_ref, h3_ref, as_ref, ad_ref):
    h = jnp.dot(x_ref[...], w_ref[...], preferred_element_type=_f32,
                precision=lax.Precision.HIGHEST)
    for g, ref in enumerate((h0_ref, h1_ref, h2_ref, h3_ref)):
        ref[...] = h[:, g * 128:(g + 1) * 128]
    _alpha_tabs(h, h.shape[0], asv_ref[...], adv_ref[...], as_ref, ad_ref)


def _norm_relu(aa_ref, ab_ref, da_ref, db_ref, b_ref):
    rd = 1.0 / (da_ref[...] + db_ref[...] + 1e-16)
    b = b_ref[...]
    bn = rd.shape[0]
    parts = []
    for g in range(G):
        acc = aa_ref[g] + ab_ref[g]
        f = jnp.concatenate(
            [jnp.broadcast_to(rd[:, 2 * g:2 * g + 1], (bn, HID)),
             jnp.broadcast_to(rd[:, 2 * g + 1:2 * g + 2], (bn, HID))], axis=1)
        parts.append(jax.nn.relu(acc * f + b[0, g * 128:(g + 1) * 128]))
    return jnp.concatenate(parts, axis=1)


def _mm23_body(aa_ref, ab_ref, da_ref, db_ref, b_ref, x_ref, wa_ref, wb_ref,
               asv_ref, adv_ref,
               h0_ref, h1_ref, h2_ref, h3_ref, as_ref, ad_ref):
    a = _norm_relu(aa_ref, ab_ref, da_ref, db_ref, b_ref)
    h = (jnp.dot(a, wa_ref[...], preferred_element_type=_f32,
                 precision=lax.Precision.HIGHEST)
         + jnp.dot(x_ref[...], wb_ref[...], preferred_element_type=_f32,
                   precision=lax.Precision.HIGHEST))
    for g, ref in enumerate((h0_ref, h1_ref, h2_ref, h3_ref)):
        ref[...] = h[:, g * 128:(g + 1) * 128]
    _alpha_tabs(h, h.shape[0], asv_ref[...], adv_ref[...], as_ref, ad_ref)


def _mm3_body(aa_ref, ab_ref, da_ref, db_ref, b_ref, x_ref, wa_ref, wb_ref,
              asv_ref, adv_ref, h_ref, as_ref, ad_ref):
    a = _norm_relu(aa_ref, ab_ref, da_ref, db_ref, b_ref)
    h = (jnp.dot(a, wa_ref[...], preferred_element_type=_f32,
                 precision=lax.Precision.HIGHEST)
         + jnp.dot(x_ref[...], wb_ref[...], preferred_element_type=_f32,
                   precision=lax.Precision.HIGHEST))
    bn = h.shape[0]
    h_ref[...] = h
    a_s = jnp.sum(h * asv_ref[...], axis=1, keepdims=True)
    a_d = jnp.sum(h * adv_ref[...], axis=1, keepdims=True)
    z = jnp.zeros((bn, H16 - 1), _f32)
    as_ref[...] = jnp.concatenate([a_s, z], axis=1)
    ad_ref[...] = jnp.concatenate([a_d, z], axis=1)


def _epi3_body(aa_ref, ab_ref, da_ref, db_ref, b_ref, o_ref):
    rd = 1.0 / (da_ref[...] + db_ref[...] + 1e-16)
    s = (aa_ref[...] + ab_ref[...]) * rd[:, 0:1] + b_ref[...]
    bn = s.shape[0]
    col = lax.broadcasted_iota(jnp.int32, (bn, H16), 1)
    valid = col < OUT
    s = jnp.where(valid, s, -1e30)
    m = jnp.max(s, axis=1, keepdims=True)
    e = jnp.where(valid, jnp.exp(s - m), 0.0)
    lse = jnp.log(jnp.sum(e, axis=1, keepdims=True))
    o_ref[...] = (s - m - lse)[:, :OUT]


_BN = 1000
_spec_n16 = pl.BlockSpec((_BN, H16), lambda i: (i, 0))
_spec_acc = pl.BlockSpec((G, _BN, 128), lambda i: (0, i, 0))


def _mm1(x, w, asv, adv):
    return pl.pallas_call(
        _mm1_body,
        grid=(N // _BN,),
        in_specs=[
            pl.BlockSpec((_BN, D_FEAT), lambda i: (i, 0)),
            pl.BlockSpec((D_FEAT, HEADS * HID), lambda i: (0, 0)),
            pl.BlockSpec((1, HEADS, HID), lambda i: (0, 0, 0)),
            pl.BlockSpec((1, HEADS, HID), lambda i: (0, 0, 0)),
        ],
        out_specs=[pl.BlockSpec((_BN, 128), lambda i: (i, 0))] * 4
        + [_spec_n16] * 2,
        out_shape=[jax.ShapeDtypeStruct((N, 128), _f32)] * 4
        + [jax.ShapeDtypeStruct((N, H16), _f32)] * 2,
    )(x, w, asv, adv)


def _mm23(aa, ab, da, db, b, x, wa, wb, asv, adv):
    return pl.pallas_call(
        _mm23_body,
        grid=(N // _BN,),
        in_specs=[
            _spec_acc, _spec_acc, _spec_n16, _spec_n16,
            pl.BlockSpec((1, HEADS * HID), lambda i: (0, 0)),
            pl.BlockSpec((_BN, D_FEAT), lambda i: (i, 0)),
            pl.BlockSpec((HEADS * HID, HEADS * HID), lambda i: (0, 0)),
            pl.BlockSpec((D_FEAT, HEADS * HID), lambda i: (0, 0)),
            pl.BlockSpec((1, HEADS, HID), lambda i: (0, 0, 0)),
            pl.BlockSpec((1, HEADS, HID), lambda i: (0, 0, 0)),
        ],
        out_specs=[pl.BlockSpec((_BN, 128), lambda i: (i, 0))] * 4
        + [_spec_n16] * 2,
        out_shape=[jax.ShapeDtypeStruct((N, 128), _f32)] * 4
        + [jax.ShapeDtypeStruct((N, H16), _f32)] * 2,
    )(aa, ab, da, db, b, x, wa, wb, asv, adv)


def _mm3(aa, ab, da, db, b, x, wa, wb, asv, adv):
    return pl.pallas_call(
        _mm3_body,
        grid=(N // _BN,),
        in_specs=[
            _spec_acc, _spec_acc, _spec_n16, _spec_n16,
            pl.BlockSpec((1, HEADS * HID), lambda i: (0, 0)),
            pl.BlockSpec((_BN, D_FEAT), lambda i: (i, 0)),
            pl.BlockSpec((HEADS * HID, H16), lambda i: (0, 0)),
            pl.BlockSpec((D_FEAT, H16), lambda i: (0, 0)),
            pl.BlockSpec((1, H16), lambda i: (0, 0)),
            pl.BlockSpec((1, H16), lambda i: (0, 0)),
        ],
        out_specs=[_spec_n16] * 3,
        out_shape=[jax.ShapeDtypeStruct((N, H16), _f32)] * 3,
    )(aa, ab, da, db, b, x, wa, wb, asv, adv)


def _epi3(aa, ab, da, db, b):
    return pl.pallas_call(
        _epi3_body,
        grid=(N // _BN,),
        in_specs=[_spec_n16, _spec_n16, _spec_n16, _spec_n16,
                  pl.BlockSpec((1, H16), lambda i: (0, 0))],
        out_specs=pl.BlockSpec((_BN, OUT), lambda i: (i, 0)),
        out_shape=jax.ShapeDtypeStruct((N, OUT), _f32),
    )(aa, ab, da, db, b)


@functools.partial(
    pl.kernel,
    out_type=(
        jax.ShapeDtypeStruct((E_PAD, H16), _f32),
        jax.ShapeDtypeStruct((NPAD, H16), _f32),
        jax.ShapeDtypeStruct((NPAD, H16), _f32),
    ),
    mesh=_mesh,
    compiler_params=_sc_params,
    scratch_types=[
        pltpu.VMEM((CH, K), jnp.int32),
        pltpu.VMEM((CH, K), jnp.int32),
        pltpu.VMEM((2, K, H16), _f32),
        pltpu.VMEM((2, K, H16), _f32),
        pltpu.VMEM((2, K, H16), _f32),
        pltpu.VMEM_SHARED((NPAD, H16), _f32),
        pltpu.SemaphoreType.DMA,
        pltpu.SemaphoreType.DMA,
        pltpu.SemaphoreType.DMA,
        pltpu.SemaphoreType.DMA,
    ],
)
def _phase1(src3_h, dst3_h, as_h, ad_h, zr16_h, ex_h, da_h, db_h,
            src_a, dst_a, as_v, ad_v, exo_v, den_sh,
            gsem0, gsem1, wsem0, wsem1):
    cid = lax.axis_index("c")
    sid = lax.axis_index("s")
    wid = sid * NC + cid
    row0 = sid * ROWS_PT

    pltpu.sync_copy(src3_h.at[wid], src_a)
    pltpu.sync_copy(dst3_h.at[wid], dst_a)
    for t in range(ROWS_PT // 128):
        pltpu.sync_copy(zr16_h, den_sh.at[pl.ds(row0 + t * 128, 128), :])
    plsc.subcore_barrier()

    def issue(j, slot, gsem):
        pltpu.async_copy(as_h.at[src_a.at[j]], as_v.at[slot], gsem)
        pltpu.async_copy(ad_h.at[dst_a.at[j]], ad_v.at[slot], gsem)

    def drain_g(slot, gsem):
        pltpu.make_async_copy(as_h.at[pl.ds(0, K)], as_v.at[slot], gsem).wait()
        pltpu.make_async_copy(ad_h.at[pl.ds(0, K)], ad_v.at[slot], gsem).wait()

    def drain_w(slot, wsem):
        pltpu.make_async_copy(exo_v.at[slot], ex_h.at[pl.ds(0, K), :], wsem).wait()

    issue(0, 0, gsem0)

    def step(i, _):
        p = i % 2
        nxt = i + 1

        @pl.when(jnp.logical_and(nxt < CH, nxt % 2 == 0))
        def _():
            drain_w(0, wsem0)
            issue(nxt, 0, gsem0)

        @pl.when(jnp.logical_and(nxt < CH, nxt % 2 == 1))
        def _():
            @pl.when(nxt >= 3)
            def _():
                drain_w(1, wsem1)

            issue(nxt, 1, gsem1)

        @pl.when(p == 0)
        def _():
            drain_g(0, gsem0)

        @pl.when(p == 1)
        def _():
            drain_g(1, gsem1)

        def edge(e, _):
            v = as_v[p, e, :] + ad_v[p, e, :]
            v = jnp.where(v > 0.0, v, 0.2 * v)
            exo_v[p, e, :] = jnp.exp(v)
            return 0

        lax.fori_loop(0, K, edge, 0, unroll=2)

        @pl.when(p == 0)
        def _():
            pltpu.async_copy(exo_v.at[0], ex_h.at[pl.ds((wid * CH + i) * K, K), :], wsem0)

        @pl.when(p == 1)
        def _():
            pltpu.async_copy(exo_v.at[1], ex_h.at[pl.ds((wid * CH + i) * K, K), :], wsem1)

        pltpu.sync_copy(exo_v.at[p], den_sh.at[dst_a.at[i]], add=True)
        return 0

    lax.fori_loop(0, CH, step, 0)
    drain_w(0, wsem0)
    drain_w(1, wsem1)
    plsc.subcore_barrier()

    @pl.when(cid == 0)
    def _():
        pltpu.sync_copy(den_sh.at[pl.ds(row0, ROWS_PT), :],
                        da_h.at[pl.ds(row0, ROWS_PT), :])

    @pl.when(cid == 1)
    def _():
        pltpu.sync_copy(den_sh.at[pl.ds(row0, ROWS_PT), :],
                        db_h.at[pl.ds(row0, ROWS_PT), :])


@functools.partial(
    pl.kernel,
    out_type=(
        jax.ShapeDtypeStruct((G, NPAD, 128), _f32),
        jax.ShapeDtypeStruct((G, NPAD, 128), _f32),
    ),
    mesh=_mesh,
    compiler_params=_sc_params,
    scratch_types=[
        pltpu.VMEM((2, K), jnp.int32),
        pltpu.VMEM((2, K), jnp.int32),
        pltpu.VMEM((2, K, H16), _f32),
        pltpu.VMEM((2, K, 128), _f32),
        pltpu.VMEM_SHARED((NPAD, 128), _f32),
        pltpu.SemaphoreType.DMA,
        pltpu.SemaphoreType.DMA,
        pltpu.SemaphoreType.DMA,
        pltpu.SemaphoreType.DMA,
    ],
)
def _phase2(src3_h, dst3_h, ex_h, zr_h, h0_h, h1_h, h2_h, h3_h, pa_h, pb_h,
            src_s, dst_s, ex_v, h_v, acc_sh, isem0, isem1, gsem0, gsem1):
    cid = lax.axis_index("c")
    sid = lax.axis_index("s")
    wid = sid * NC + cid
    row0 = sid * ROWS_PT

    def idx_issue(j, slot, isem):
        pltpu.async_copy(src3_h.at[wid, j], src_s.at[slot], isem)
        pltpu.async_copy(dst3_h.at[wid, j], dst_s.at[slot], isem)

    def idx_drain(slot, isem):
        pltpu.make_async_copy(src3_h.at[0, 0], src_s.at[slot], isem).wait()
        pltpu.make_async_copy(dst3_h.at[0, 0], dst_s.at[slot], isem).wait()

    for g, h_tab in enumerate((h0_h, h1_h, h2_h, h3_h)):
        for t in range(ROWS_PT // 128):
            pltpu.sync_copy(zr_h, acc_sh.at[pl.ds(row0 + t * 128, 128), :])
        plsc.subcore_barrier()

        def issue(j, slot, gsem):
            pltpu.async_copy(
                ex_h.at[pl.ds((wid * CH + j) * K, K), :], ex_v.at[slot], gsem)
            pltpu.async_copy(h_tab.at[src_s.at[slot]], h_v.at[slot], gsem)

        def drain_g(slot, gsem):
            pltpu.make_async_copy(ex_h.at[pl.ds(0, K), :], ex_v.at[slot],
                                  gsem).wait()
            pltpu.make_async_copy(h_tab.at[pl.ds(0, K)], h_v.at[slot],
                                  gsem).wait()

        idx_issue(0, 0, isem0)
        idx_issue(1, 1, isem1)
        idx_drain(0, isem0)
        issue(0, 0, gsem0)

        def step(i, _):
            p = i % 2
            nxt = i + 1

            @pl.when(jnp.logical_and(nxt < CH, nxt % 2 == 0))
            def _():
                idx_drain(0, isem0)
                issue(nxt, 0, gsem0)

            @pl.when(jnp.logical_and(nxt < CH, nxt % 2 == 1))
            def _():
                idx_drain(1, isem1)
                issue(nxt, 1, gsem1)

            @pl.when(p == 0)
            def _():
                drain_g(0, gsem0)

            @pl.when(p == 1)
            def _():
                drain_g(1, gsem1)

            def edge(e, _):
                exr = ex_v[p, e, :]
                c0 = exr[2 * g]
                c1 = exr[2 * g + 1]
                for j in range(4):
                    h_v[p, e, pl.ds(j * 16, 16)] = \
                        h_v[p, e, pl.ds(j * 16, 16)] * c0
                for j in range(4, 8):
                    h_v[p, e, pl.ds(j * 16, 16)] = \
                        h_v[p, e, pl.ds(j * 16, 16)] * c1
                return 0

            lax.fori_loop(0, K, edge, 0, unroll=2)
            pltpu.sync_copy(h_v.at[p], acc_sh.at[dst_s.at[p]], add=True)

            @pl.when(jnp.logical_and(i + 2 < CH, p == 0))
            def _():
                idx_issue(i + 2, 0, isem0)

            @pl.when(jnp.logical_and(i + 2 < CH, p == 1))
            def _():
                idx_issue(i + 2, 1, isem1)

            return 0

        lax.fori_loop(0, CH, step, 0)
        plsc.subcore_barrier()

        @pl.when(cid == 0)
        def _():
            pltpu.sync_copy(acc_sh.at[pl.ds(row0, ROWS_PT), :],
                            pa_h.at[g, pl.ds(row0, ROWS_PT), :])

        @pl.when(cid == 1)
        def _():
            pltpu.sync_copy(acc_sh.at[pl.ds(row0, ROWS_PT), :],
                            pb_h.at[g, pl.ds(row0, ROWS_PT), :])

        plsc.subcore_barrier()


@functools.partial(
    pl.kernel,
    out_type=(
        jax.ShapeDtypeStruct((NPAD, H16), _f32),
        jax.ShapeDtypeStruct((NPAD, H16), _f32),
    ),
    mesh=_mesh,
    compiler_params=_sc_params,
    scratch_types=[
        pltpu.VMEM((CH, K), jnp.int32),
        pltpu.VMEM((CH, K), jnp.int32),
        pltpu.VMEM((2, K, H16), _f32),
        pltpu.VMEM((2, K, H16), _f32),
        pltpu.VMEM_SHARED((NPAD, H16), _f32),
        pltpu.SemaphoreType.DMA,
        pltpu.SemaphoreType.DMA,
    ],
)
def _phase2s(src3_h, dst3_h, ex_h, zr16_h, ht_h, pa_h, pb_h,
             src_a, dst_a, ex_v, h_v, acc_sh, gsem0, gsem1):
    cid = lax.axis_index("c")
    sid = lax.axis_index("s")
    wid = sid * NC + cid
    row0 = sid * ROWS_PT

    pltpu.sync_copy(src3_h.at[wid], src_a)
    pltpu.sync_copy(dst3_h.at[wid], dst_a)
    for t in range(ROWS_PT // 128):
        pltpu.sync_copy(zr16_h, acc_sh.at[pl.ds(row0 + t * 128, 128), :])
    plsc.subcore_barrier()

    def issue(j, slot, gsem):
        pltpu.async_copy(ex_h.at[pl.ds((wid * CH + j) * K, K), :], ex_v.at[slot], gsem)
        pltpu.async_copy(ht_h.at[src_a.at[j]], h_v.at[slot], gsem)

    def drain_g(slot, gsem):
        pltpu.make_async_copy(ex_h.at[pl.ds(0, K), :], ex_v.at[slot], gsem).wait()
        pltpu.make_async_copy(ht_h.at[pl.ds(0, K)], h_v.at[slot], gsem).wait()

    issue(0, 0, gsem0)

    def step(i, _):
        p = i % 2
        nxt = i + 1

        @pl.when(jnp.logical_and(nxt < CH, nxt % 2 == 0))
        def _():
            issue(nxt, 0, gsem0)

        @pl.when(jnp.logical_and(nxt < CH, nxt % 2 == 1))
        def _():
            issue(nxt, 1, gsem1)

        @pl.when(p == 0)
        def _():
            drain_g(0, gsem0)

        @pl.when(p == 1)
        def _():
            drain_g(1, gsem1)

        def edge(e, _):
            exr = ex_v[p, e, :]
            h_v[p, e, :] = h_v[p, e, :] * exr[0]
            return 0

        lax.fori_loop(0, K, edge, 0, unroll=2)
        pltpu.sync_copy(h_v.at[p], acc_sh.at[dst_a.at[i]], add=True)
        return 0

    lax.fori_loop(0, CH, step, 0)
    plsc.subcore_barrier()

    @pl.when(cid == 0)
    def _():
        pltpu.sync_copy(acc_sh.at[pl.ds(row0, ROWS_PT), :],
                        pa_h.at[pl.ds(row0, ROWS_PT), :])

    @pl.when(cid == 1)
    def _():
        pltpu.sync_copy(acc_sh.at[pl.ds(row0, ROWS_PT), :],
                        pb_h.at[pl.ds(row0, ROWS_PT), :])


def _pad_tab(t):
    return jnp.concatenate([t, jnp.zeros((NPAD - N, H16), _f32)], axis=0)


def kernel(x, edge_index, W1, att_src1, att_dst1, b1,
           W2, att_src2, att_dst2, b2, W3, att_src3, att_dst3, b3):
    loop = jnp.arange(N, dtype=edge_index.dtype)
    npad_e = E_PAD - E_TOT
    src = jnp.concatenate(
        [edge_index[0], loop,
         jnp.zeros((npad_e,), edge_index.dtype)]).reshape(NW, CH, K)
    dst = jnp.concatenate(
        [edge_index[1], loop,
         jnp.full((npad_e,), N, edge_index.dtype)]).reshape(NW, CH, K)

    zr = jnp.zeros((128, 128), _f32)
    zr16 = jnp.zeros((128, H16), _f32)

    h0, h1, h2, h3, a_s, a_d = _mm1(x, W1, att_src1, att_dst1)
    ex, da, db = _phase1(src, dst, _pad_tab(a_s), _pad_tab(a_d), zr16)
    pa, pb = _phase2(src, dst, ex, zr, h0, h1, h2, h3)

    h0, h1, h2, h3, a_s, a_d = _mm23(
        pa, pb, da, db, b1.reshape(1, -1), x, W2[:HEADS * HID],
        W2[HEADS * HID:], att_src2, att_dst2)
    ex, da, db = _phase1(src, dst, _pad_tab(a_s), _pad_tab(a_d), zr16)
    pa, pb = _phase2(src, dst, ex, zr, h0, h1, h2, h3)

    w3p = jnp.concatenate([W3, jnp.zeros((W3.shape[0], H16 - OUT), _f32)],
                          axis=1)
    a3s = jnp.concatenate(
        [att_src3.reshape(1, OUT), jnp.zeros((1, H16 - OUT), _f32)], axis=1)
    a3d = jnp.concatenate(
        [att_dst3.reshape(1, OUT), jnp.zeros((1, H16 - OUT), _f32)], axis=1)
    ht, a_s, a_d = _mm3(
        pa, pb, da, db, b2.reshape(1, -1), x, w3p[:HEADS * HID],
        w3p[HEADS * HID:], a3s, a3d)
    ex, da, db = _phase1(src, dst, _pad_tab(a_s), _pad_tab(a_d), zr16)
    pa3, pb3 = _phase2s(src, dst, ex, zr16, ht)

    b3p = jnp.concatenate(
        [b3.reshape(1, OUT), jnp.zeros((1, H16 - OUT), _f32)], axis=1)
    return _epi3(pa3, pb3, da, db, b3p)
